# Initial kernel scaffold; baseline (speedup 1.0000x reference)
import jax
import jax.numpy as jnp
from jax.experimental import pallas as pl


def kernel(features0, features1, type_mask, edge0, dst0, target_idx0, edge1, dst1, target_idx1, adjM, idx_batch, W_fc0, b_fc0, W_fc1, b_fc1, attn0, attn1, mp_fc1_w, mp_fc1_b, mp_fc2_w, layer_fc_w, layer_fc_b, nc_fc1_w, nc_fc1_b, nc_fc2_w, final_w, final_b):
    raise NotImplementedError("write your pallas kernel here")


# speedup vs baseline: 8.5525x; 8.5525x over previous
import dataclasses
import functools

import jax
import jax.numpy as jnp
from jax import lax
from jax.experimental import pallas as pl
from jax.experimental.pallas import tpu as pltpu
from jax.experimental.pallas import tpu_sc as plsc

N_NODES = 10000
N_TYPE = 5000
B = 2048
K = 16
E = 160000
HID = 64
ATT = 32

NC = 2
NS = 16
NW = NC * NS
CHUNK = 128
E_PAD = 163840
NCHUNK = E_PAD // CHUNK
CPT = NCHUNK // NW
ACC_W = 80
ACC_R = 2176
RPT = ACC_R // NS
DST_PAD = B

_GDN = lax.GatherDimensionNumbers(
    offset_dims=(), collapsed_slice_dims=(0,), start_index_map=(0,))

_SC_PARAMS = pltpu.CompilerParams(
    needs_layout_passes=False, use_tc_tiling_on_sc=False)


def _xform_body(x_ref, w_ref, b_ref, a0_ref, a1_ref, t_ref, s_ref):
    t = jnp.dot(x_ref[...], w_ref[...].T, preferred_element_type=jnp.float32)
    t = t + b_ref[...]
    t_ref[...] = t
    s0 = jnp.sum(t * a0_ref[...], axis=1, keepdims=True)
    s1 = jnp.sum(t * a1_ref[...], axis=1, keepdims=True)
    s_ref[...] = jnp.concatenate([s0, s1], axis=1) * (1.0 / 3.0)


def _xform(x, w, b, attn0, attn1):
    return pl.pallas_call(
        _xform_body,
        out_shape=(
            jax.ShapeDtypeStruct((N_TYPE, HID), jnp.float32),
            jax.ShapeDtypeStruct((N_TYPE, 2), jnp.float32),
        ),
    )(x, w, b, attn0, attn1)


def _edge_sc_body(t_hbm, et0_hbm, dst0_hbm, et1_hbm, dst1_hbm, s0_hbm, s1_hbm,
                  zeros_hbm, out_hbm,
                  etb, dstb, rows, wrows, svm, acc0, acc1, sem):
    cid = lax.axis_index("c")
    sid = lax.axis_index("s")
    wid = cid * NS + sid
    iota = lax.iota(jnp.int32, 16)
    one_hot0 = jnp.where(iota == 0, 1.0, 0.0).astype(jnp.float32)

    @pl.when(sid == 0)
    def _():
        pltpu.sync_copy(zeros_hbm, acc0)
        pltpu.sync_copy(zeros_hbm, acc1)

    plsc.subcore_barrier()

    for m, (et_hbm, dst_hbm, s_hbm, acc) in enumerate(
            ((et0_hbm, dst0_hbm, s0_hbm, acc0),
             (et1_hbm, dst1_hbm, s1_hbm, acc1))):
        pltpu.sync_copy(s_hbm, svm)

        @pl.loop(0, CPT)
        def _(c):
            g = wid * CPT + c
            pltpu.sync_copy(et_hbm.at[g], etb)
            pltpu.sync_copy(dst_hbm.at[g], dstb)
            cps = [pltpu.async_copy(t_hbm.at[etb.at[l]],
                                    rows.at[pl.ds(l * CHUNK, CHUNK)], sem)
                   for l in range(3)]
            for cp in cps:
                cp.wait()
            for grp in range(CHUNK // 16):
                gsl = pl.ds(grp * 16, 16)
                a = (plsc.load_gather(svm, [etb[0, gsl]])
                     + plsc.load_gather(svm, [etb[1, gsl]])
                     + plsc.load_gather(svm, [etb[2, gsl]]))
                a = jnp.where(a >= 0.0, a, a * 0.01)
                ae = jnp.exp(a)
                ae3 = ae * (1.0 / 3.0)

                @pl.loop(0, 16)
                def _(i):
                    e = grp * 16 + i
                    bidx = jnp.broadcast_to(i, (16, 1)).astype(jnp.int32)
                    w3 = lax.gather(ae3, bidx, _GDN, (1,),
                                    mode=lax.GatherScatterMode.PROMISE_IN_BOUNDS)
                    w1 = lax.gather(ae, bidx, _GDN, (1,),
                                    mode=lax.GatherScatterMode.PROMISE_IN_BOUNDS)
                    for f in range(HID // 16):
                        fsl = pl.ds(f * 16, 16)
                        v = (rows[e, fsl] + rows[CHUNK + e, fsl]
                             + rows[2 * CHUNK + e, fsl])
                        wrows[e, fsl] = v * w3
                    wrows[e, pl.ds(HID, 16)] = w1 * one_hot0

            pltpu.sync_copy(wrows, acc.at[dstb], add=True)

    plsc.subcore_barrier()
    rsl = pl.ds(sid * RPT, RPT)
    pltpu.sync_copy(acc0.at[rsl], out_hbm.at[0, cid, rsl])
    pltpu.sync_copy(acc1.at[rsl], out_hbm.at[1, cid, rsl])


def _edge_sc(t, et0, dst0, et1, dst1, s0, s1, zeros):
    mesh = plsc.VectorSubcoreMesh(core_axis_name="c", subcore_axis_name="s")
    fn = functools.partial(
        pl.kernel,
        out_type=jax.ShapeDtypeStruct((2, NC, ACC_R, ACC_W), jnp.float32),
        mesh=mesh,
        compiler_params=_SC_PARAMS,
        scratch_types=[
            pltpu.VMEM((3, CHUNK), jnp.int32),
            pltpu.VMEM((CHUNK,), jnp.int32),
            pltpu.VMEM((3 * CHUNK, HID), jnp.float32),
            pltpu.VMEM((CHUNK, ACC_W), jnp.float32),
            pltpu.VMEM((N_NODES,), jnp.float32),
            pltpu.VMEM_SHARED((ACC_R, ACC_W), jnp.float32),
            pltpu.VMEM_SHARED((ACC_R, ACC_W), jnp.float32),
            pltpu.SemaphoreType.DMA,
        ],
    )(_edge_sc_body)
    return fn(t, et0, dst0, et1, dst1, s0, s1, zeros)


def _adj_sc_body(adj_hbm, t_hbm, out_hbm, arow, posbuf, nrows, sem):
    cid = lax.axis_index("c")
    sid = lax.axis_index("s")
    wid = cid * NS + sid
    iota = lax.iota(jnp.int32, 16)
    rows_per_tile = B // NW

    @pl.loop(0, rows_per_tile)
    def _(rr):
        r = wid * rows_per_tile + rr
        pltpu.sync_copy(adj_hbm.at[r], arow)

        def scan_body(j, off):
            v = arow[pl.ds(j * 16, 16)]
            msk = v > 0
            pos = iota + j * 16
            plsc.store_compressed(posbuf.at[pl.ds(off, 16)], pos, mask=msk)
            return off + jnp.max(plsc.all_reduce_population_count(msk))

        lax.fori_loop(0, N_NODES // 16, scan_body, jnp.int32(0))
        nidx = posbuf[pl.ds(0, 16)]
        pltpu.async_copy(t_hbm.at[nidx], nrows, sem).wait()
        pltpu.sync_copy(nrows, out_hbm.at[pl.ds(r * K, K)])


def _adj_sc(adj, t):
    mesh = plsc.VectorSubcoreMesh(core_axis_name="c", subcore_axis_name="s")
    fn = functools.partial(
        pl.kernel,
        out_type=jax.ShapeDtypeStruct((B * K, HID), jnp.float32),
        mesh=mesh,
        compiler_params=_SC_PARAMS,
        scratch_types=[
            pltpu.VMEM((N_NODES,), jnp.int32),
            pltpu.VMEM((32,), jnp.int32),
            pltpu.VMEM((K, HID), jnp.float32),
            pltpu.SemaphoreType.DMA,
        ],
    )(_adj_sc_body)
    return fn(adj, t)


def _beta_body(acc_ref, mp1w_ref, mp1b_ref, mp2w_ref,
               mp0_ref, mp1_ref, beta_ref):
    acc = acc_ref[...]

    def metapath(m):
        num = acc[m, 0, :B, :HID] + acc[m, 1, :B, :HID]
        den = acc[m, 0, :B, HID] + acc[m, 1, :B, HID]
        nf = num / den[:, None]
        return jnp.where(nf > 0, nf, jnp.exp(nf) - 1.0)

    mp0 = metapath(0)
    mp1 = metapath(1)
    mp0_ref[...] = mp0
    mp1_ref[...] = mp1
    mp1w = mp1w_ref[...]
    mp1b = mp1b_ref[...]
    mp2w = mp2w_ref[...]

    def beta_fn(m):
        f1 = jnp.tanh(jnp.dot(m, mp1w.T,
                              preferred_element_type=jnp.float32) + mp1b)
        return jnp.dot(jnp.mean(f1, axis=0, keepdims=True), mp2w.T,
                       preferred_element_type=jnp.float32)[0]

    beta = jnp.concatenate([beta_fn(mp0), beta_fn(mp1)], axis=0)
    beta_ref[...] = jax.nn.softmax(beta, axis=0)


def _attn_body(mp0_ref, mp1_ref, beta_ref, nbr_ref,
               lw_ref, lb_ref, nc1w_ref, nc1b_ref, nc2w_ref,
               fw_ref, fb_ref, out_ref, hp_ref):
    beta = beta_ref[...]
    h = beta[0] * mp0_ref[...] + beta[1] * mp1_ref[...]
    hfc = jnp.dot(h, lw_ref[...].T,
                  preferred_element_type=jnp.float32) + lb_ref[...]
    h2 = jnp.where(hfc > 0, hfc, jnp.exp(hfc) - 1.0)

    nc1w = nc1w_ref[...]
    nc1b = nc1b_ref[...]
    nc2w = nc2w_ref[...]
    nbr = nbr_ref[...]

    def score(x):
        f1 = jnp.tanh(jnp.dot(x, nc1w.T,
                              preferred_element_type=jnp.float32) + nc1b)
        return jnp.dot(f1, nc2w.T, preferred_element_type=jnp.float32)

    scores = jnp.concatenate([score(h2)] + [score(nbr[k]) for k in range(K)],
                             axis=1)
    gamma = jax.nn.softmax(scores, axis=1)
    hp = gamma[:, 0:1] * h2
    for k in range(K):
        hp = hp + gamma[:, k + 1:k + 2] * nbr[k]
    hp_ref[...] = hp
    out_ref[...] = jnp.dot(hp, fw_ref[...].T,
                           preferred_element_type=jnp.float32) + fb_ref[...]


_BLK = 256


def _combine(acc, nbr, mp_fc1_w, mp_fc1_b, mp_fc2_w, layer_fc_w, layer_fc_b,
             nc_fc1_w, nc_fc1_b, nc_fc2_w, final_w, final_b):
    mp0, mp1, beta = pl.pallas_call(
        _beta_body,
        out_shape=(
            jax.ShapeDtypeStruct((B, HID), jnp.float32),
            jax.ShapeDtypeStruct((B, HID), jnp.float32),
            jax.ShapeDtypeStruct((2,), jnp.float32),
        ),
    )(acc, mp_fc1_w, mp_fc1_b, mp_fc2_w)

    nbr2 = nbr.reshape(B, K, HID).transpose(1, 0, 2)
    grid = (B // _BLK,)
    bspec = pl.BlockSpec((_BLK, HID), lambda i: (i, 0))
    wfull = lambda *s: pl.BlockSpec(s, lambda i: tuple(0 for _ in s))
    out, hp = pl.pallas_call(
        _attn_body,
        grid=grid,
        in_specs=[
            bspec, bspec, wfull(2),
            pl.BlockSpec((K, _BLK, HID), lambda i: (0, i, 0)),
            wfull(HID, HID), wfull(HID), wfull(ATT, HID), wfull(ATT),
            wfull(1, ATT), wfull(HID, HID), wfull(HID),
        ],
        out_specs=(bspec, bspec),
        out_shape=(
            jax.ShapeDtypeStruct((B, HID), jnp.float32),
            jax.ShapeDtypeStruct((B, HID), jnp.float32),
        ),
    )(mp0, mp1, beta, nbr2, layer_fc_w, layer_fc_b,
      nc_fc1_w, nc_fc1_b, nc_fc2_w, final_w, final_b)
    return out, hp


def _prep_edges(edge, dst):
    pe = jnp.concatenate(
        [edge, jnp.zeros((E_PAD - E, 3), jnp.int32)], axis=0)
    pd = jnp.concatenate(
        [dst, jnp.full((E_PAD - E,), DST_PAD, jnp.int32)], axis=0)
    et = pe.reshape(NCHUNK, CHUNK, 3).transpose(0, 2, 1)
    return et, pd.reshape(NCHUNK, CHUNK)


def kernel(features0, features1, type_mask, edge0, dst0, target_idx0,
           edge1, dst1, target_idx1, adjM, idx_batch,
           W_fc0, b_fc0, W_fc1, b_fc1, attn0, attn1,
           mp_fc1_w, mp_fc1_b, mp_fc2_w, layer_fc_w, layer_fc_b,
           nc_fc1_w, nc_fc1_b, nc_fc2_w, final_w, final_b):
    t0, sA = _xform(features0, W_fc0, b_fc0, attn0, attn1)
    t1, sB = _xform(features1, W_fc1, b_fc1, attn0, attn1)
    t = jnp.concatenate([t0, t1], axis=0)
    s0 = jnp.concatenate([sA[:, 0], sB[:, 0]], axis=0)
    s1 = jnp.concatenate([sA[:, 1], sB[:, 1]], axis=0)

    et0, dp0 = _prep_edges(edge0, dst0)
    et1, dp1 = _prep_edges(edge1, dst1)
    zeros = jnp.zeros((ACC_R, ACC_W), jnp.float32)

    acc = _edge_sc(t, et0, dp0, et1, dp1, s0, s1, zeros)
    nbr = _adj_sc(adjM, t)

    return _combine(acc, nbr, mp_fc1_w, mp_fc1_b, mp_fc2_w,
                    layer_fc_w, layer_fc_b, nc_fc1_w, nc_fc1_b, nc_fc2_w,
                    final_w, final_b)


# speedup vs baseline: 10.3492x; 1.2101x over previous
import dataclasses
import functools

import jax
import jax.numpy as jnp
from jax import lax
from jax.experimental import pallas as pl
from jax.experimental.pallas import tpu as pltpu
from jax.experimental.pallas import tpu_sc as plsc

N_NODES = 10000
N_TYPE = 5000
B = 2048
K = 16
E = 160000
HID = 64
ATT = 32

NC = 2
NS = 16
NW = NC * NS
CHUNK = 128
E_PAD = 163840
NCHUNK = E_PAD // CHUNK
CPT = NCHUNK // NW
ACC_W = 80
ACC_R = 2176
RPT = ACC_R // NS
DST_PAD = B

_GDN = lax.GatherDimensionNumbers(
    offset_dims=(), collapsed_slice_dims=(0,), start_index_map=(0,))

_SC_PARAMS = pltpu.CompilerParams(
    needs_layout_passes=False, use_tc_tiling_on_sc=False)


def _xform_body(x_ref, w_ref, b_ref, a0_ref, a1_ref, t_ref, s_ref):
    t = jnp.dot(x_ref[...], w_ref[...].T, preferred_element_type=jnp.float32)
    t = t + b_ref[...]
    t_ref[...] = t
    s0 = jnp.sum(t * a0_ref[...], axis=1, keepdims=True)
    s1 = jnp.sum(t * a1_ref[...], axis=1, keepdims=True)
    s_ref[...] = jnp.concatenate([s0, s1], axis=1) * (1.0 / 3.0)


def _xform(x, w, b, attn0, attn1):
    return pl.pallas_call(
        _xform_body,
        out_shape=(
            jax.ShapeDtypeStruct((N_TYPE, HID), jnp.float32),
            jax.ShapeDtypeStruct((N_TYPE, 2), jnp.float32),
        ),
    )(x, w, b, attn0, attn1)


def _edge_sc_body(t_hbm, et0_hbm, dst0_hbm, et1_hbm, dst1_hbm, s0_hbm, s1_hbm,
                  zeros_hbm, out_hbm,
                  etall, dstall, rows0, rows1, wrows, svm, acc0, acc1,
                  sem0, sem1):
    cid = lax.axis_index("c")
    sid = lax.axis_index("s")
    wid = cid * NS + sid
    iota = lax.iota(jnp.int32, 16)
    one_hot0 = jnp.where(iota == 0, 1.0, 0.0).astype(jnp.float32)

    @pl.when(sid == 0)
    def _():
        pltpu.sync_copy(zeros_hbm, acc0)
        pltpu.sync_copy(zeros_hbm, acc1)

    plsc.subcore_barrier()

    def gathers(c, rows, sem):
        return [pltpu.async_copy(t_hbm.at[etall.at[c, l]],
                                 rows.at[pl.ds(l * CHUNK, CHUNK)], sem)
                for l in range(3)]

    def compute(c, rows):
        @pl.loop(0, CHUNK // 16)
        def _(grp):
            gsl = pl.ds(grp * 16, 16)
            a = (plsc.load_gather(svm, [etall[c, 0, gsl]])
                 + plsc.load_gather(svm, [etall[c, 1, gsl]])
                 + plsc.load_gather(svm, [etall[c, 2, gsl]]))
            a = jnp.where(a >= 0.0, a, a * 0.01)
            ae3 = jnp.exp(a) * (1.0 / 3.0)
            e0 = grp * 16
            for i in range(16):
                e = e0 + i
                bidx = jnp.full((16, 1), i, jnp.int32)
                w3 = lax.gather(ae3, bidx, _GDN, (1,),
                                mode=lax.GatherScatterMode.PROMISE_IN_BOUNDS)
                for f in range(HID // 16):
                    fsl = pl.ds(f * 16, 16)
                    v = (rows[e, fsl] + rows[CHUNK + e, fsl]
                         + rows[2 * CHUNK + e, fsl])
                    wrows[e, fsl] = v * w3
                wrows[e, pl.ds(HID, 16)] = (w3 * 3.0) * one_hot0

    for m, (et_hbm, dst_hbm, s_hbm, acc) in enumerate(
            ((et0_hbm, dst0_hbm, s0_hbm, acc0),
             (et1_hbm, dst1_hbm, s1_hbm, acc1))):
        pltpu.sync_copy(s_hbm, svm)
        pltpu.sync_copy(et_hbm.at[pl.ds(wid * CPT, CPT)], etall)
        pltpu.sync_copy(dst_hbm.at[pl.ds(wid * CPT, CPT)], dstall)

        def wait3(c, rows, sem):
            for l in range(3):
                pltpu.make_async_copy(t_hbm.at[etall.at[c, l]],
                                      rows.at[pl.ds(l * CHUNK, CHUNK)],
                                      sem).wait()

        gathers(0, rows0, sem0)

        @pl.loop(0, CPT // 2)
        def _(k):
            c0 = 2 * k
            c1 = c0 + 1
            gathers(c1, rows1, sem1)
            wait3(c0, rows0, sem0)
            compute(c0, rows0)
            pltpu.sync_copy(wrows, acc.at[dstall.at[c0]], add=True)

            @pl.when(c1 + 1 < CPT)
            def _():
                gathers(c1 + 1, rows0, sem0)

            wait3(c1, rows1, sem1)
            compute(c1, rows1)
            pltpu.sync_copy(wrows, acc.at[dstall.at[c1]], add=True)

    plsc.subcore_barrier()
    rsl = pl.ds(sid * RPT, RPT)
    pltpu.sync_copy(acc0.at[rsl], out_hbm.at[0, cid, rsl])
    pltpu.sync_copy(acc1.at[rsl], out_hbm.at[1, cid, rsl])


def _edge_sc(t, et0, dst0, et1, dst1, s0, s1, zeros):
    mesh = plsc.VectorSubcoreMesh(core_axis_name="c", subcore_axis_name="s")
    fn = functools.partial(
        pl.kernel,
        out_type=jax.ShapeDtypeStruct((2, NC, ACC_R, ACC_W), jnp.float32),
        mesh=mesh,
        compiler_params=_SC_PARAMS,
        scratch_types=[
            pltpu.VMEM((CPT, 3, CHUNK), jnp.int32),
            pltpu.VMEM((CPT, CHUNK), jnp.int32),
            pltpu.VMEM((3 * CHUNK, HID), jnp.float32),
            pltpu.VMEM((3 * CHUNK, HID), jnp.float32),
            pltpu.VMEM((CHUNK, ACC_W), jnp.float32),
            pltpu.VMEM((N_NODES,), jnp.float32),
            pltpu.VMEM_SHARED((ACC_R, ACC_W), jnp.float32),
            pltpu.VMEM_SHARED((ACC_R, ACC_W), jnp.float32),
            pltpu.SemaphoreType.DMA,
            pltpu.SemaphoreType.DMA,
        ],
    )(_edge_sc_body)
    return fn(t, et0, dst0, et1, dst1, s0, s1, zeros)


def _adj_sc_body(adj_hbm, t_hbm, out_hbm, arow, posbuf, nrows, sem):
    cid = lax.axis_index("c")
    sid = lax.axis_index("s")
    wid = cid * NS + sid
    iota = lax.iota(jnp.int32, 16)
    rows_per_tile = B // NW

    @pl.loop(0, rows_per_tile)
    def _(rr):
        r = wid * rows_per_tile + rr
        pltpu.sync_copy(adj_hbm.at[r], arow)

        def scan_body(j, off):
            v = arow[pl.ds(j * 16, 16)]
            msk = v > 0
            pos = iota + j * 16
            plsc.store_compressed(posbuf.at[pl.ds(off, 16)], pos, mask=msk)
            return off + jnp.max(plsc.all_reduce_population_count(msk))

        lax.fori_loop(0, N_NODES // 16, scan_body, jnp.int32(0), unroll=8)
        nidx = posbuf[pl.ds(0, 16)]
        pltpu.async_copy(t_hbm.at[nidx], nrows, sem).wait()
        pltpu.sync_copy(nrows, out_hbm.at[pl.ds(r * K, K)])


def _adj_sc(adj, t):
    mesh = plsc.VectorSubcoreMesh(core_axis_name="c", subcore_axis_name="s")
    fn = functools.partial(
        pl.kernel,
        out_type=jax.ShapeDtypeStruct((B * K, HID), jnp.float32),
        mesh=mesh,
        compiler_params=_SC_PARAMS,
        scratch_types=[
            pltpu.VMEM((N_NODES,), jnp.int32),
            pltpu.VMEM((32,), jnp.int32),
            pltpu.VMEM((K, HID), jnp.float32),
            pltpu.SemaphoreType.DMA,
        ],
    )(_adj_sc_body)
    return fn(adj, t)


def _beta_body(acc_ref, mp1w_ref, mp1b_ref, mp2w_ref,
               mp0_ref, mp1_ref, beta_ref):
    acc = acc_ref[...]

    def metapath(m):
        num = acc[m, 0, :B, :HID] + acc[m, 1, :B, :HID]
        den = acc[m, 0, :B, HID] + acc[m, 1, :B, HID]
        nf = num / den[:, None]
        return jnp.where(nf > 0, nf, jnp.exp(nf) - 1.0)

    mp0 = metapath(0)
    mp1 = metapath(1)
    mp0_ref[...] = mp0
    mp1_ref[...] = mp1
    mp1w = mp1w_ref[...]
    mp1b = mp1b_ref[...]
    mp2w = mp2w_ref[...]

    def beta_fn(m):
        f1 = jnp.tanh(jnp.dot(m, mp1w.T,
                              preferred_element_type=jnp.float32) + mp1b)
        return jnp.dot(jnp.mean(f1, axis=0, keepdims=True), mp2w.T,
                       preferred_element_type=jnp.float32)[0]

    beta = jnp.concatenate([beta_fn(mp0), beta_fn(mp1)], axis=0)
    beta_ref[...] = jax.nn.softmax(beta, axis=0)


def _attn_body(mp0_ref, mp1_ref, beta_ref, nbr_ref,
               lw_ref, lb_ref, nc1w_ref, nc1b_ref, nc2w_ref,
               fw_ref, fb_ref, out_ref, hp_ref):
    beta = beta_ref[...]
    h = beta[0] * mp0_ref[...] + beta[1] * mp1_ref[...]
    hfc = jnp.dot(h, lw_ref[...].T,
                  preferred_element_type=jnp.float32) + lb_ref[...]
    h2 = jnp.where(hfc > 0, hfc, jnp.exp(hfc) - 1.0)

    nc1w = nc1w_ref[...]
    nc1b = nc1b_ref[...]
    nc2w = nc2w_ref[...]
    nbr = nbr_ref[...]

    def score(x):
        f1 = jnp.tanh(jnp.dot(x, nc1w.T,
                              preferred_element_type=jnp.float32) + nc1b)
        return jnp.dot(f1, nc2w.T, preferred_element_type=jnp.float32)

    scores = jnp.concatenate([score(h2)] + [score(nbr[k]) for k in range(K)],
                             axis=1)
    gamma = jax.nn.softmax(scores, axis=1)
    hp = gamma[:, 0:1] * h2
    for k in range(K):
        hp = hp + gamma[:, k + 1:k + 2] * nbr[k]
    hp_ref[...] = hp
    out_ref[...] = jnp.dot(hp, fw_ref[...].T,
                           preferred_element_type=jnp.float32) + fb_ref[...]


_BLK = 256


def _combine(acc, nbr, mp_fc1_w, mp_fc1_b, mp_fc2_w, layer_fc_w, layer_fc_b,
             nc_fc1_w, nc_fc1_b, nc_fc2_w, final_w, final_b):
    mp0, mp1, beta = pl.pallas_call(
        _beta_body,
        out_shape=(
            jax.ShapeDtypeStruct((B, HID), jnp.float32),
            jax.ShapeDtypeStruct((B, HID), jnp.float32),
            jax.ShapeDtypeStruct((2,), jnp.float32),
        ),
    )(acc, mp_fc1_w, mp_fc1_b, mp_fc2_w)

    nbr2 = nbr.reshape(B, K, HID).transpose(1, 0, 2)
    grid = (B // _BLK,)
    bspec = pl.BlockSpec((_BLK, HID), lambda i: (i, 0))
    wfull = lambda *s: pl.BlockSpec(s, lambda i: tuple(0 for _ in s))
    out, hp = pl.pallas_call(
        _attn_body,
        grid=grid,
        in_specs=[
            bspec, bspec, wfull(2),
            pl.BlockSpec((K, _BLK, HID), lambda i: (0, i, 0)),
            wfull(HID, HID), wfull(HID), wfull(ATT, HID), wfull(ATT),
            wfull(1, ATT), wfull(HID, HID), wfull(HID),
        ],
        out_specs=(bspec, bspec),
        out_shape=(
            jax.ShapeDtypeStruct((B, HID), jnp.float32),
            jax.ShapeDtypeStruct((B, HID), jnp.float32),
        ),
    )(mp0, mp1, beta, nbr2, layer_fc_w, layer_fc_b,
      nc_fc1_w, nc_fc1_b, nc_fc2_w, final_w, final_b)
    return out, hp


def _prep_edges(edge, dst):
    pe = jnp.concatenate(
        [edge, jnp.zeros((E_PAD - E, 3), jnp.int32)], axis=0)
    pd = jnp.concatenate(
        [dst, jnp.full((E_PAD - E,), DST_PAD, jnp.int32)], axis=0)
    et = pe.reshape(NCHUNK, CHUNK, 3).transpose(0, 2, 1)
    return et, pd.reshape(NCHUNK, CHUNK)


def kernel(features0, features1, type_mask, edge0, dst0, target_idx0,
           edge1, dst1, target_idx1, adjM, idx_batch,
           W_fc0, b_fc0, W_fc1, b_fc1, attn0, attn1,
           mp_fc1_w, mp_fc1_b, mp_fc2_w, layer_fc_w, layer_fc_b,
           nc_fc1_w, nc_fc1_b, nc_fc2_w, final_w, final_b):
    t0, sA = _xform(features0, W_fc0, b_fc0, attn0, attn1)
    t1, sB = _xform(features1, W_fc1, b_fc1, attn0, attn1)
    t = jnp.concatenate([t0, t1], axis=0)
    s0 = jnp.concatenate([sA[:, 0], sB[:, 0]], axis=0)
    s1 = jnp.concatenate([sA[:, 1], sB[:, 1]], axis=0)

    et0, dp0 = _prep_edges(edge0, dst0)
    et1, dp1 = _prep_edges(edge1, dst1)
    zeros = jnp.zeros((ACC_R, ACC_W), jnp.float32)

    acc = _edge_sc(t, et0, dp0, et1, dp1, s0, s1, zeros)
    nbr = _adj_sc(adjM, t)

    return _combine(acc, nbr, mp_fc1_w, mp_fc1_b, mp_fc2_w,
                    layer_fc_w, layer_fc_b, nc_fc1_w, nc_fc1_b, nc_fc2_w,
                    final_w, final_b)
